# Initial kernel scaffold; baseline (speedup 1.0000x reference)
import jax
import jax.numpy as jnp
from jax.experimental import pallas as pl


def kernel(choices, rewards, alpha_plus, alpha_minus, initial_values):
    raise NotImplementedError("write your pallas kernel here")


# speedup vs baseline: 11.5414x; 11.5414x over previous
import functools

import jax
import jax.numpy as jnp
from jax import lax
from jax.experimental import pallas as pl
from jax.experimental.pallas import tpu as pltpu
from jax.experimental.pallas import tpu_sc as plsc

_N, _T, _K = 4096, 200, 32
_LANES = 16
_NW = 32
_TPW = _N // _NW
_NG = _TPW // _LANES
_MAX_IV = 100.0


def _rw_body(ch_hbm, rw_hbm, par_hbm, out_hbm, ch_v, rw_v, par_v, buf,
             sem_a, sem_b):
    wid = lax.axis_index("s") * 2 + lax.axis_index("c")
    base = wid * _TPW

    pltpu.sync_copy(ch_hbm.at[:, pl.ds(base, _TPW)], ch_v)
    pltpu.sync_copy(rw_hbm.at[:, pl.ds(base, _TPW)], rw_v)
    pltpu.sync_copy(par_hbm, par_v)

    lanes = lax.iota(jnp.int32, _LANES)
    one = jnp.full((_LANES,), 1.0, jnp.float32)
    ap = one / (one + jnp.exp(-par_v[0, :]))
    am = one / (one + jnp.exp(-par_v[1, :]))
    e2 = jnp.exp(par_v[2, :] * 2.0)
    iv = _MAX_IV * (e2 - one) / (e2 + one)

    def init_body(tk, acc):
        for h in range(2):
            buf[tk, pl.ds(h * _LANES, _LANES)] = iv
        return acc

    lax.fori_loop(0, 2 * _TPW, init_body, 0)

    def out_row(t):
        return out_hbm.at[pl.ds(base, _TPW), t]

    def snap(slot):
        return buf.at[pl.ds(slot * _TPW, _TPW)]

    pltpu.make_async_copy(snap(0), out_row(0), sem_a).start()

    def row_body(t, carry):
        pidx, pval = carry
        slot = lax.rem(t, 2)
        srow = slot * _TPW

        @pl.when(slot == 0)
        def _():
            pltpu.make_async_copy(snap(0), out_row(t), sem_a).wait()

        @pl.when(jnp.logical_and(slot == 1, t >= 3))
        def _():
            pltpu.make_async_copy(snap(1), out_row(t), sem_b).wait()

        nidx = []
        nval = []
        for g in range(_NG):
            trow = lanes + (g * _LANES) + srow
            plsc.store_scatter(buf, [trow, pidx[g]], pval[g])
            ch = ch_v[t - 1, pl.ds(g * _LANES, _LANES)]
            rw = rw_v[t - 1, pl.ds(g * _LANES, _LANES)]
            chosen = plsc.load_gather(buf, [trow, ch])
            pe = rw - chosen
            pe = jnp.where(rw != rw, jnp.zeros_like(pe), pe)
            coef = jnp.where(pe >= 0.0, ap, am)
            val = chosen + coef * pe
            plsc.store_scatter(buf, [trow, ch], val)
            nidx.append(ch)
            nval.append(val)

        @pl.when(slot == 0)
        def _():
            pltpu.make_async_copy(snap(0), out_row(t), sem_a).start()

        @pl.when(slot == 1)
        def _():
            pltpu.make_async_copy(snap(1), out_row(t), sem_b).start()

        return tuple(nidx), tuple(nval)

    carry0 = (tuple(lanes for _ in range(_NG)),
              tuple(iv for _ in range(_NG)))
    lax.fori_loop(1, _T, row_body, carry0)

    pltpu.make_async_copy(snap(0), out_row(0), sem_a).wait()
    pltpu.make_async_copy(snap(1), out_row(1), sem_b).wait()


_rw_kernel = functools.partial(
    pl.kernel,
    out_type=jax.ShapeDtypeStruct((_N, _T, _K), jnp.float32),
    mesh=plsc.VectorSubcoreMesh(core_axis_name="c", subcore_axis_name="s"),
    compiler_params=pltpu.CompilerParams(needs_layout_passes=False),
    scratch_types=[
        pltpu.VMEM((_T, _TPW), jnp.int32),
        pltpu.VMEM((_T, _TPW), jnp.float32),
        pltpu.VMEM((3, _LANES), jnp.float32),
        pltpu.VMEM((2 * _TPW, _K), jnp.float32),
        pltpu.SemaphoreType.DMA,
        pltpu.SemaphoreType.DMA,
    ],
)(_rw_body)


def kernel(choices, rewards, alpha_plus, alpha_minus, initial_values):
    chT = jnp.transpose(choices).astype(jnp.int32)
    rwT = jnp.transpose(rewards).astype(jnp.float32)
    par = jnp.stack([alpha_plus, alpha_minus, initial_values])
    par = jnp.broadcast_to(par.astype(jnp.float32)[:, None], (3, _LANES))
    return _rw_kernel(chT, rwT, par)
